# baseline (speedup 1.0000x reference)
import jax
import jax.numpy as jnp
from jax.experimental import pallas as pl
from jax.experimental.pallas import tpu as pltpu

_N = 10000
_D = 256
_C = 64

_TM = 200
_NBUF = 4
_NSLICE = 5
_TS = _TM // _NSLICE
_NBLK = _N // _TM


def _gcn_kernel(x_hbm, adj_hbm, w1_ref, b1_ref, wmt_ref, bm_ref, out_ref,
                abuf, xbuf, sup, sem, xsem):
    i = pl.program_id(0)

    def slice_copy(blk, s):
        return pltpu.make_async_copy(
            adj_hbm.at[pl.ds(blk * _TM + s * _TS, _TS), :],
            abuf.at[blk % _NBUF, pl.ds(s * _TS, _TS), :],
            sem.at[blk % _NBUF, s],
        )

    def x_copy():
        return pltpu.make_async_copy(x_hbm, xbuf, xsem)

    @pl.when(i == 0)
    def _():
        x_copy().start()
        for blk in range(_NBUF - 1):
            for s in range(_NSLICE):
                slice_copy(blk, s).start()
        x_copy().wait()
        sup[...] = jnp.dot(xbuf[...], w1_ref[...],
                           preferred_element_type=jnp.float32
                           ).astype(jnp.bfloat16)

    @pl.when((i >= 1) & (i + _NBUF - 2 < _NBLK))
    def _():
        for s in range(_NSLICE):
            slice_copy(i + _NBUF - 2, s).start()

    @pl.when(i >= 1)
    def _():
        b = i - 1
        for s in range(_NSLICE):
            slice_copy(b, s).wait()
        h = jnp.dot(abuf[b % _NBUF].astype(jnp.bfloat16), sup[...],
                    preferred_element_type=jnp.float32)
        h = jnp.maximum(h + b1_ref[...], 0.0)
        out_ref[...] = jnp.dot(
            h, wmt_ref[...], preferred_element_type=jnp.float32,
        ) + bm_ref[...]


def kernel(x, adj, W1, b1, W_mlp, b_mlp):
    wmt = W_mlp.T
    b1_2d = b1.reshape(1, _D)
    bm_2d = b_mlp.reshape(1, _C)

    out = pl.pallas_call(
        _gcn_kernel,
        grid=(_NBLK + 1,),
        in_specs=[
            pl.BlockSpec(memory_space=pl.ANY),
            pl.BlockSpec(memory_space=pl.ANY),
            pl.BlockSpec((_D, _D), lambda i: (0, 0)),
            pl.BlockSpec((1, _D), lambda i: (0, 0)),
            pl.BlockSpec((_D, _C), lambda i: (0, 0)),
            pl.BlockSpec((1, _C), lambda i: (0, 0)),
        ],
        out_specs=pl.BlockSpec(
            (_TM, _C), lambda i: (jnp.maximum(i - 1, 0), 0)),
        out_shape=jax.ShapeDtypeStruct((_N, _C), jnp.float32),
        scratch_shapes=[
            pltpu.VMEM((_NBUF, _TM, _N), jnp.float32),
            pltpu.VMEM((_N, _D), jnp.float32),
            pltpu.VMEM((_N, _D), jnp.bfloat16),
            pltpu.SemaphoreType.DMA((_NBUF, _NSLICE)),
            pltpu.SemaphoreType.DMA,
        ],
        compiler_params=pltpu.CompilerParams(
            dimension_semantics=("arbitrary",),
            vmem_limit_bytes=100 * 1024 * 1024,
        ),
    )(x, adj, W1, b1_2d, wmt, bm_2d)
    return out


# speedup vs baseline: 1.0290x; 1.0290x over previous
import jax
import jax.numpy as jnp
from jax.experimental import pallas as pl
from jax.experimental.pallas import tpu as pltpu

_N = 10000
_D = 256
_C = 64

_TM = 200
_NBUF = 3
_NSLICE = 1
_TS = _TM // _NSLICE
_NBLK = _N // _TM


def _gcn_kernel(x_hbm, adj_hbm, w1_ref, b1_ref, wmt_ref, bm_ref, out_ref,
                abuf, xbuf, sup, sem, xsem):
    i = pl.program_id(0)

    def slice_copy(blk, s):
        return pltpu.make_async_copy(
            adj_hbm.at[pl.ds(blk * _TM + s * _TS, _TS), :],
            abuf.at[blk % _NBUF, pl.ds(s * _TS, _TS), :],
            sem.at[blk % _NBUF, s],
        )

    def x_copy():
        return pltpu.make_async_copy(x_hbm, xbuf, xsem)

    @pl.when(i == 0)
    def _():
        x_copy().start()
        for blk in range(_NBUF - 1):
            for s in range(_NSLICE):
                slice_copy(blk, s).start()
        x_copy().wait()
        sup[...] = jnp.dot(xbuf[...], w1_ref[...],
                           preferred_element_type=jnp.float32
                           ).astype(jnp.bfloat16)

    @pl.when((i >= 1) & (i + _NBUF - 2 < _NBLK))
    def _():
        for s in range(_NSLICE):
            slice_copy(i + _NBUF - 2, s).start()

    @pl.when(i >= 1)
    def _():
        b = i - 1
        for s in range(_NSLICE):
            slice_copy(b, s).wait()
        h = jnp.dot(abuf[b % _NBUF].astype(jnp.bfloat16), sup[...],
                    preferred_element_type=jnp.float32)
        h = jnp.maximum(h + b1_ref[...], 0.0)
        out_ref[...] = jnp.dot(
            h, wmt_ref[...], preferred_element_type=jnp.float32,
        ) + bm_ref[...]


def kernel(x, adj, W1, b1, W_mlp, b_mlp):
    wmt = W_mlp.T
    b1_2d = b1.reshape(1, _D)
    bm_2d = b_mlp.reshape(1, _C)

    out = pl.pallas_call(
        _gcn_kernel,
        grid=(_NBLK + 1,),
        in_specs=[
            pl.BlockSpec(memory_space=pl.ANY),
            pl.BlockSpec(memory_space=pl.ANY),
            pl.BlockSpec((_D, _D), lambda i: (0, 0)),
            pl.BlockSpec((1, _D), lambda i: (0, 0)),
            pl.BlockSpec((_D, _C), lambda i: (0, 0)),
            pl.BlockSpec((1, _C), lambda i: (0, 0)),
        ],
        out_specs=pl.BlockSpec(
            (_TM, _C), lambda i: (jnp.maximum(i - 1, 0), 0)),
        out_shape=jax.ShapeDtypeStruct((_N, _C), jnp.float32),
        scratch_shapes=[
            pltpu.VMEM((_NBUF, _TM, _N), jnp.float32),
            pltpu.VMEM((_N, _D), jnp.float32),
            pltpu.VMEM((_N, _D), jnp.bfloat16),
            pltpu.SemaphoreType.DMA((_NBUF, _NSLICE)),
            pltpu.SemaphoreType.DMA,
        ],
        compiler_params=pltpu.CompilerParams(
            dimension_semantics=("arbitrary",),
            vmem_limit_bytes=100 * 1024 * 1024,
        ),
    )(x, adj, W1, b1_2d, wmt, bm_2d)
    return out
